# baseline (speedup 1.0000x reference)
import functools

import jax
import jax.numpy as jnp
from jax import lax
from jax.experimental import pallas as pl
from jax.experimental.pallas import tpu as pltpu
from jax.experimental.pallas import tpu_sc as plsc

_MAX_LEN = 200
_D_MODEL = 64
_BATCH = 4096
_ROW = _MAX_LEN * _D_MODEL

_NUM_CORES = 2
_NUM_SUBCORES = 16
_NUM_WORKERS = _NUM_CORES * _NUM_SUBCORES
_ROWS_PER_W = _BATCH // _NUM_WORKERS

_REP_S = 64
_REP_PER_TEC = _REP_S // _NUM_SUBCORES
_ROWS_PER_SC = _BATCH // _NUM_CORES
_CHUNKS = _ROWS_PER_W // _REP_S


@functools.partial(
    pl.kernel,
    out_type=jax.ShapeDtypeStruct((_BATCH, _ROW), jnp.float32),
    mesh=plsc.VectorSubcoreMesh(core_axis_name="c", subcore_axis_name="s"),
    scratch_types=[
        pltpu.VMEM((_ROW,), jnp.float32),
        pltpu.VMEM_SHARED((_REP_S, _ROW), jnp.float32),
        pltpu.SemaphoreType.DMA,
    ],
)
def _pe_broadcast(w_hbm, out_hbm, tbuf, shared, sem):
    c = lax.axis_index("c")
    s = lax.axis_index("s")
    pltpu.sync_copy(w_hbm, tbuf)
    for r in range(_REP_PER_TEC):
        pltpu.sync_copy(tbuf, shared.at[s * _REP_PER_TEC + r])
    plsc.subcore_barrier()
    base = c * _ROWS_PER_SC + s * _ROWS_PER_W
    copies = [
        pltpu.async_copy(shared, out_hbm.at[pl.ds(base + i * _REP_S, _REP_S)], sem)
        for i in range(_CHUNKS)
    ]
    for cp in copies:
        cp.wait()


def kernel(x, pe_weight):
    del x
    out = _pe_broadcast(pe_weight.reshape(_ROW))
    return out.reshape(_BATCH, _MAX_LEN, _D_MODEL)
